# baseline (speedup 1.0000x reference)
import functools
import jax
import jax.numpy as jnp
from jax import lax
from jax.experimental import pallas as pl
from jax.experimental.pallas import tpu as pltpu
from jax.experimental.pallas import tpu_sc as plsc

BN = 512
BT = 1024
BE = 640
NW = 32
CH_A = 128
CH_B = 64


def _sc_mesh():
    return plsc.VectorSubcoreMesh(core_axis_name="c", subcore_axis_name="s")


def _make_gather_rows(R, H, M):
    rpw = M // NW
    nch = rpw // CH_A

    @functools.partial(
        pl.kernel,
        out_type=jax.ShapeDtypeStruct((M, H), jnp.float32),
        mesh=_sc_mesh(),
        scratch_types=[
            pltpu.VMEM((nch, CH_A), jnp.int32),
            pltpu.VMEM((CH_A, H), jnp.float32),
            pltpu.SemaphoreType.DMA,
        ],
    )
    def gather_rows(table_hbm, idx_hbm, out_hbm, idx_v, buf_v, sem):
        wid = lax.axis_index("s") * 2 + lax.axis_index("c")
        base = wid * rpw
        pltpu.sync_copy(idx_hbm.at[wid], idx_v)
        for ck in range(nch):
            pltpu.async_copy(table_hbm.at[idx_v.at[ck]], buf_v, sem).wait()
            pltpu.sync_copy(buf_v, out_hbm.at[pl.ds(base + ck * CH_A, CH_A)])

    return gather_rows


def _make_step_gather(EPAD, H, NP, E):
    rpw = NP // NW
    nch = rpw // CH_B
    nv = rpw // 16

    @functools.partial(
        pl.kernel,
        out_type=jax.ShapeDtypeStruct((NP, H), jnp.float32),
        mesh=_sc_mesh(),
        scratch_types=[
            pltpu.VMEM((rpw,), jnp.int32),
            pltpu.VMEM((rpw,), jnp.int32),
            pltpu.VMEM((nch, CH_B), jnp.int32),
            pltpu.VMEM((CH_B, H), jnp.float32),
            pltpu.VMEM((16,), jnp.int32),
            pltpu.SemaphoreType.DMA,
        ],
    )
    def step_gather(feats_hbm, starts_hbm, counts_hbm, tk_hbm, xt_hbm,
                    sv, cv, sidx, buf_v, tk_v, sem):
        wid = lax.axis_index("s") * 2 + lax.axis_index("c")
        base = wid * rpw
        pltpu.sync_copy(tk_hbm, tk_v)
        pltpu.sync_copy(starts_hbm.at[pl.ds(base, rpw)], sv)
        pltpu.sync_copy(counts_hbm.at[pl.ds(base, rpw)], cv)
        t16 = tk_v[...]
        for v in range(nv):
            s16 = sv[pl.ds(v * 16, 16)]
            c16 = cv[pl.ds(v * 16, 16)]
            ge = jnp.minimum(s16 + t16, E - 1)
            sidx[v // (CH_B // 16), pl.ds((v % (CH_B // 16)) * 16, 16)] = (
                jnp.where(c16 > t16, ge, E))
        for ck in range(nch):
            pltpu.async_copy(feats_hbm.at[sidx.at[ck]], buf_v, sem).wait()
            pltpu.sync_copy(
                buf_v, xt_hbm.at[pl.ds(base + ck * CH_B, CH_B)])

    return step_gather


def _lstm_step_body(kt_ref, xt_ref, h_ref, c_ref, wih_ref, whh_ref, b_ref,
                    h_out, c_out, gates_ref):
    i = pl.program_id(0)
    hdim = h_ref.shape[1]
    gates_ref[...] = (
        jnp.dot(h_ref[...], whh_ref[...], preferred_element_type=jnp.float32)
        + b_ref[...]
    )

    @pl.when(i * BN < kt_ref[0])
    def _():
        gates_ref[...] += jnp.dot(xt_ref[...], wih_ref[...],
                                  preferred_element_type=jnp.float32)

    g = gates_ref[...]
    gi = jax.nn.sigmoid(g[:, 0 * hdim:1 * hdim])
    gf = jax.nn.sigmoid(g[:, 1 * hdim:2 * hdim])
    gg = jnp.tanh(g[:, 2 * hdim:3 * hdim])
    go = jax.nn.sigmoid(g[:, 3 * hdim:4 * hdim])
    c_new = gf * c_ref[...] + gi * gg
    h_out[...] = go * jnp.tanh(c_new)
    c_out[...] = c_new


def _make_lstm_step(NP, H):
    NB = NP // BN

    def xt_map(i, kt):
        last = jnp.maximum(pl.cdiv(kt[0], BN) - 1, 0)
        return (jnp.minimum(i, last), 0)

    grid_spec = pltpu.PrefetchScalarGridSpec(
        num_scalar_prefetch=1,
        grid=(NB,),
        in_specs=[
            pl.BlockSpec((BN, H), xt_map),
            pl.BlockSpec((BN, H), lambda i, kt: (i, 0)),
            pl.BlockSpec((BN, H), lambda i, kt: (i, 0)),
            pl.BlockSpec((H, 4 * H), lambda i, kt: (0, 0)),
            pl.BlockSpec((H, 4 * H), lambda i, kt: (0, 0)),
            pl.BlockSpec((1, 4 * H), lambda i, kt: (0, 0)),
        ],
        out_specs=[
            pl.BlockSpec((BN, H), lambda i, kt: (i, 0)),
            pl.BlockSpec((BN, H), lambda i, kt: (i, 0)),
        ],
        scratch_shapes=[pltpu.VMEM((BN, 4 * H), jnp.float32)],
    )
    return pl.pallas_call(
        _lstm_step_body,
        grid_spec=grid_spec,
        out_shape=[
            jax.ShapeDtypeStruct((NP, H), jnp.float32),
            jax.ShapeDtypeStruct((NP, H), jnp.float32),
        ],
        compiler_params=pltpu.CompilerParams(
            dimension_semantics=("arbitrary",)),
    )


def _tail_body(aggr_ref, h_ref, wl_ref, wr_ref, b_ref, o_ref, *, nvalid):
    i = pl.program_id(0)
    v = (jnp.dot(aggr_ref[...], wl_ref[...], preferred_element_type=jnp.float32)
         + jnp.dot(h_ref[...], wr_ref[...], preferred_element_type=jnp.float32)
         + b_ref[...])
    v = jnp.maximum(v, 0.0)
    rows = i * BT + lax.broadcasted_iota(jnp.int32, v.shape, 0)
    o_ref[...] = jnp.where(rows < nvalid, v, 0.0)


def _make_tail(NP, H, N):
    return pl.pallas_call(
        functools.partial(_tail_body, nvalid=N),
        grid=(NP // BT,),
        in_specs=[
            pl.BlockSpec((BT, H), lambda i: (i, 0)),
            pl.BlockSpec((BT, H), lambda i: (i, 0)),
            pl.BlockSpec((H, H), lambda i: (0, 0)),
            pl.BlockSpec((H, H), lambda i: (0, 0)),
            pl.BlockSpec((1, H), lambda i: (0, 0)),
        ],
        out_specs=pl.BlockSpec((BT, H), lambda i: (i, 0)),
        out_shape=jax.ShapeDtypeStruct((NP, H), jnp.float32),
        compiler_params=pltpu.CompilerParams(
            dimension_semantics=("arbitrary",)),
    )


def _mlp_body(hs_ref, hd_ref, ea_ref, w1s_ref, w1d_ref, w1e_ref, b1_ref,
              w2_ref, b2_ref, w3_ref, b3_ref, o_ref):
    z = (jnp.dot(hs_ref[...], w1s_ref[...], preferred_element_type=jnp.float32)
         + jnp.dot(hd_ref[...], w1d_ref[...], preferred_element_type=jnp.float32)
         + jnp.dot(ea_ref[...], w1e_ref[...], preferred_element_type=jnp.float32)
         + b1_ref[...])
    z = jnp.maximum(z, 0.0)
    z = jnp.maximum(
        jnp.dot(z, w2_ref[...], preferred_element_type=jnp.float32)
        + b2_ref[...], 0.0)
    o_ref[...] = (jnp.dot(z, w3_ref[...], preferred_element_type=jnp.float32)
                  + b3_ref[...])


def _make_mlp(EP, H, ED, H2, OUT):
    return pl.pallas_call(
        _mlp_body,
        grid=(EP // BE,),
        in_specs=[
            pl.BlockSpec((BE, H), lambda i: (i, 0)),
            pl.BlockSpec((BE, H), lambda i: (i, 0)),
            pl.BlockSpec((BE, ED), lambda i: (i, 0)),
            pl.BlockSpec((H, H), lambda i: (0, 0)),
            pl.BlockSpec((H, H), lambda i: (0, 0)),
            pl.BlockSpec((ED, H), lambda i: (0, 0)),
            pl.BlockSpec((1, H), lambda i: (0, 0)),
            pl.BlockSpec((H, H2), lambda i: (0, 0)),
            pl.BlockSpec((1, H2), lambda i: (0, 0)),
            pl.BlockSpec((H2, OUT), lambda i: (0, 0)),
            pl.BlockSpec((1, OUT), lambda i: (0, 0)),
        ],
        out_specs=pl.BlockSpec((BE, OUT), lambda i: (i, 0)),
        out_shape=jax.ShapeDtypeStruct((EP, OUT), jnp.float32),
        compiler_params=pltpu.CompilerParams(
            dimension_semantics=("arbitrary",)),
    )


def kernel(x, edge_index, edge_attr, params):
    x = x.astype(jnp.float32)
    src = edge_index[0].astype(jnp.int32)
    dst = edge_index[1].astype(jnp.int32)
    N, D = x.shape
    E = src.shape[0]
    H = D
    NP = -(-N // 2560) * 2560
    EPAD = -(-E // (NW * CH_A * 2)) * (NW * CH_A * 2)
    EP = EPAD
    assert EP % BE == 0

    starts_all = jnp.searchsorted(
        dst, jnp.arange(N, dtype=jnp.int32)).astype(jnp.int32)
    counts = jnp.diff(jnp.concatenate(
        [starts_all, jnp.array([E], jnp.int32)]))
    T = counts.max().astype(jnp.int32)

    order = jnp.argsort(-counts).astype(jnp.int32)
    counts_s = jnp.concatenate(
        [counts[order], jnp.zeros((NP - N,), jnp.int32)])
    starts_s = jnp.concatenate(
        [starts_all[order], jnp.full((NP - N,), E - 1, jnp.int32)])
    counts_asc = counts_s[::-1]
    pos = jnp.argsort(order).astype(jnp.int32)
    pos_src = pos[src]
    pos_dst = pos[dst]
    x_s = jnp.concatenate([x[order], jnp.zeros((NP - N, D), jnp.float32)])

    def pad_idx(ix):
        return jnp.concatenate(
            [ix, jnp.full((EPAD - E,), N, jnp.int32)]
        ).reshape(NW, EPAD // NW // CH_A, CH_A)

    idx_src = pad_idx(pos_src)
    idx_dst = pad_idx(pos_dst)

    gather_rows = _make_gather_rows(NP, H, EPAD)
    step_gather = _make_step_gather(EPAD, H, NP, E)
    lstm_step = _make_lstm_step(NP, H)
    tail = _make_tail(NP, H, N)

    def layer(h_in, p):
        wihT = p['W_ih'].T
        whhT = p['W_hh'].T
        b = (p['b_ih'] + p['b_hh']).reshape(1, 4 * H)

        feats = gather_rows(h_in, idx_src)

        def cond(carry):
            t, _, _ = carry
            return t < T

        def body(carry):
            t, h, c = carry
            kt = (NP - jnp.searchsorted(counts_asc, t, side='right')
                  ).astype(jnp.int32)
            tk = jnp.full((16,), t, jnp.int32)
            xt = step_gather(feats, starts_s, counts_s, tk)
            h, c = lstm_step(kt.reshape(1), xt, h, c, wihT, whhT, b)
            return t + 1, h, c

        z = jnp.zeros((NP, H), jnp.float32)
        _, hl, _ = lax.while_loop(cond, body, (jnp.int32(0), z, z))
        return tail(hl, h_in, p['W_l'].T, p['W_r'].T, p['b_l'].reshape(1, H))

    h = layer(x_s, params['conv1'])
    h = layer(h, params['conv2'])
    h = layer(h, params['conv3'])

    hs = gather_rows(h, idx_src)
    hd = gather_rows(h, idx_dst)
    m = params['edge_mlp']
    H2 = m['W2'].shape[0]
    OUT = m['W3'].shape[0]
    ED = edge_attr.shape[1]
    w1 = m['W1'].T

    ea = jnp.concatenate(
        [edge_attr.astype(jnp.float32),
         jnp.zeros((EP - E, ED), jnp.float32)])

    mlp = _make_mlp(EP, H, ED, H2, OUT)
    out = mlp(hs, hd, ea, w1[:H], w1[H:2 * H], w1[2 * H:],
              m['b1'].reshape(1, H),
              m['W2'].T, m['b2'].reshape(1, H2),
              m['W3'].T, m['b3'].reshape(1, OUT))
    return out[:E]


# speedup vs baseline: 1.0145x; 1.0145x over previous
import functools
import jax
import jax.numpy as jnp
from jax import lax
from jax.experimental import pallas as pl
from jax.experimental.pallas import tpu as pltpu
from jax.experimental.pallas import tpu_sc as plsc

BN = 512
BT = 1024
BE = 640
NW = 32
CH_A = 128
CH_B = 64


def _sc_mesh():
    return plsc.VectorSubcoreMesh(core_axis_name="c", subcore_axis_name="s")


def _make_gather_rows(R, H, M):
    rpw = M // NW
    nch = rpw // CH_A
    NBUF = 3

    @functools.partial(
        pl.kernel,
        out_type=jax.ShapeDtypeStruct((M, H), jnp.float32),
        mesh=_sc_mesh(),
        scratch_types=(
            [pltpu.VMEM((nch, CH_A), jnp.int32)]
            + [pltpu.VMEM((CH_A,), jnp.int32) for _ in range(NBUF)]
            + [pltpu.VMEM((CH_A, H), jnp.float32) for _ in range(NBUF)]
            + [pltpu.SemaphoreType.DMA for _ in range(NBUF)]
        ),
    )
    def gather_rows(table_hbm, idx_hbm, out_hbm, idx_v, *rest):
        idxb = rest[:NBUF]
        bufs = rest[NBUF:2 * NBUF]
        sems = rest[2 * NBUF:3 * NBUF]
        wid = lax.axis_index("s") * 2 + lax.axis_index("c")
        base = wid * rpw
        pltpu.sync_copy(idx_hbm.at[wid], idx_v)

        def fire(ck, b):
            for v in range(CH_A // 16):
                idxb[b][pl.ds(v * 16, 16)] = idx_v[ck, pl.ds(v * 16, 16)]
            pltpu.make_async_copy(
                table_hbm.at[idxb[b]], bufs[b], sems[b]).start()

        for b in range(NBUF):
            fire(b, b)
        for ck in range(nch):
            b = ck % NBUF
            pltpu.make_async_copy(
                table_hbm.at[idxb[b]], bufs[b], sems[b]).wait()
            pltpu.sync_copy(
                bufs[b], out_hbm.at[pl.ds(base + ck * CH_A, CH_A)])
            if ck + NBUF < nch:
                fire(ck + NBUF, b)

    return gather_rows


def _make_step_gather(EPAD, H, NP, E):
    rpw = NP // NW
    nch = rpw // CH_B
    nv = rpw // 16

    @functools.partial(
        pl.kernel,
        out_type=jax.ShapeDtypeStruct((NP, H), jnp.float32),
        mesh=_sc_mesh(),
        scratch_types=[
            pltpu.VMEM((rpw,), jnp.int32),
            pltpu.VMEM((rpw,), jnp.int32),
            pltpu.VMEM((rpw,), jnp.int32),
            pltpu.VMEM((rpw, H), jnp.float32),
            pltpu.VMEM((16,), jnp.int32),
            pltpu.SemaphoreType.DMA,
        ],
    )
    def step_gather(feats_hbm, starts_hbm, counts_hbm, tk_hbm, xt_hbm,
                    sv, cv, sidx, buf_v, tk_v, sem):
        wid = lax.axis_index("s") * 2 + lax.axis_index("c")
        base = wid * rpw
        pltpu.sync_copy(tk_hbm, tk_v)
        pltpu.sync_copy(starts_hbm.at[pl.ds(base, rpw)], sv)
        pltpu.sync_copy(counts_hbm.at[pl.ds(base, rpw)], cv)
        t16 = tk_v[...]
        for v in range(nv):
            s16 = sv[pl.ds(v * 16, 16)]
            c16 = cv[pl.ds(v * 16, 16)]
            ge = jnp.minimum(s16 + t16, E - 1)
            sidx[pl.ds(v * 16, 16)] = jnp.where(c16 > t16, ge, E)
        pltpu.async_copy(feats_hbm.at[sidx], buf_v, sem).wait()
        pltpu.sync_copy(buf_v, xt_hbm.at[pl.ds(base, rpw)])

    return step_gather


def _lstm_step_body(kt_ref, xt_ref, h_ref, c_ref, wih_ref, whh_ref, b_ref,
                    h_out, c_out, gates_ref):
    i = pl.program_id(0)
    hdim = h_ref.shape[1]
    gates_ref[...] = (
        jnp.dot(h_ref[...], whh_ref[...], preferred_element_type=jnp.float32)
        + b_ref[...]
    )

    @pl.when(i * BN < kt_ref[0])
    def _():
        gates_ref[...] += jnp.dot(xt_ref[...], wih_ref[...],
                                  preferred_element_type=jnp.float32)

    g = gates_ref[...]
    gi = jax.nn.sigmoid(g[:, 0 * hdim:1 * hdim])
    gf = jax.nn.sigmoid(g[:, 1 * hdim:2 * hdim])
    gg = jnp.tanh(g[:, 2 * hdim:3 * hdim])
    go = jax.nn.sigmoid(g[:, 3 * hdim:4 * hdim])
    c_new = gf * c_ref[...] + gi * gg
    h_out[...] = go * jnp.tanh(c_new)
    c_out[...] = c_new


def _make_lstm_step(NP, H):
    NB = NP // BN

    def xt_map(i, kt):
        last = jnp.maximum(pl.cdiv(kt[0], BN) - 1, 0)
        return (jnp.minimum(i, last), 0)

    grid_spec = pltpu.PrefetchScalarGridSpec(
        num_scalar_prefetch=1,
        grid=(NB,),
        in_specs=[
            pl.BlockSpec((BN, H), xt_map),
            pl.BlockSpec((BN, H), lambda i, kt: (i, 0)),
            pl.BlockSpec((BN, H), lambda i, kt: (i, 0)),
            pl.BlockSpec((H, 4 * H), lambda i, kt: (0, 0)),
            pl.BlockSpec((H, 4 * H), lambda i, kt: (0, 0)),
            pl.BlockSpec((1, 4 * H), lambda i, kt: (0, 0)),
        ],
        out_specs=[
            pl.BlockSpec((BN, H), lambda i, kt: (i, 0)),
            pl.BlockSpec((BN, H), lambda i, kt: (i, 0)),
        ],
        scratch_shapes=[pltpu.VMEM((BN, 4 * H), jnp.float32)],
    )
    return pl.pallas_call(
        _lstm_step_body,
        grid_spec=grid_spec,
        out_shape=[
            jax.ShapeDtypeStruct((NP, H), jnp.float32),
            jax.ShapeDtypeStruct((NP, H), jnp.float32),
        ],
        compiler_params=pltpu.CompilerParams(
            dimension_semantics=("arbitrary",)),
    )


def _tail_body(aggr_ref, h_ref, wl_ref, wr_ref, b_ref, o_ref, *, nvalid):
    i = pl.program_id(0)
    v = (jnp.dot(aggr_ref[...], wl_ref[...], preferred_element_type=jnp.float32)
         + jnp.dot(h_ref[...], wr_ref[...], preferred_element_type=jnp.float32)
         + b_ref[...])
    v = jnp.maximum(v, 0.0)
    rows = i * BT + lax.broadcasted_iota(jnp.int32, v.shape, 0)
    o_ref[...] = jnp.where(rows < nvalid, v, 0.0)


def _make_tail(NP, H, N):
    return pl.pallas_call(
        functools.partial(_tail_body, nvalid=N),
        grid=(NP // BT,),
        in_specs=[
            pl.BlockSpec((BT, H), lambda i: (i, 0)),
            pl.BlockSpec((BT, H), lambda i: (i, 0)),
            pl.BlockSpec((H, H), lambda i: (0, 0)),
            pl.BlockSpec((H, H), lambda i: (0, 0)),
            pl.BlockSpec((1, H), lambda i: (0, 0)),
        ],
        out_specs=pl.BlockSpec((BT, H), lambda i: (i, 0)),
        out_shape=jax.ShapeDtypeStruct((NP, H), jnp.float32),
        compiler_params=pltpu.CompilerParams(
            dimension_semantics=("arbitrary",)),
    )


def _mlp_body(hs_ref, hd_ref, ea_ref, w1s_ref, w1d_ref, w1e_ref, b1_ref,
              w2_ref, b2_ref, w3_ref, b3_ref, o_ref):
    z = (jnp.dot(hs_ref[...], w1s_ref[...], preferred_element_type=jnp.float32)
         + jnp.dot(hd_ref[...], w1d_ref[...], preferred_element_type=jnp.float32)
         + jnp.dot(ea_ref[...], w1e_ref[...], preferred_element_type=jnp.float32)
         + b1_ref[...])
    z = jnp.maximum(z, 0.0)
    z = jnp.maximum(
        jnp.dot(z, w2_ref[...], preferred_element_type=jnp.float32)
        + b2_ref[...], 0.0)
    o_ref[...] = (jnp.dot(z, w3_ref[...], preferred_element_type=jnp.float32)
                  + b3_ref[...])


def _make_mlp(EP, H, ED, H2, OUT):
    return pl.pallas_call(
        _mlp_body,
        grid=(EP // BE,),
        in_specs=[
            pl.BlockSpec((BE, H), lambda i: (i, 0)),
            pl.BlockSpec((BE, H), lambda i: (i, 0)),
            pl.BlockSpec((BE, ED), lambda i: (i, 0)),
            pl.BlockSpec((H, H), lambda i: (0, 0)),
            pl.BlockSpec((H, H), lambda i: (0, 0)),
            pl.BlockSpec((ED, H), lambda i: (0, 0)),
            pl.BlockSpec((1, H), lambda i: (0, 0)),
            pl.BlockSpec((H, H2), lambda i: (0, 0)),
            pl.BlockSpec((1, H2), lambda i: (0, 0)),
            pl.BlockSpec((H2, OUT), lambda i: (0, 0)),
            pl.BlockSpec((1, OUT), lambda i: (0, 0)),
        ],
        out_specs=pl.BlockSpec((BE, OUT), lambda i: (i, 0)),
        out_shape=jax.ShapeDtypeStruct((EP, OUT), jnp.float32),
        compiler_params=pltpu.CompilerParams(
            dimension_semantics=("arbitrary",)),
    )


def kernel(x, edge_index, edge_attr, params):
    x = x.astype(jnp.float32)
    src = edge_index[0].astype(jnp.int32)
    dst = edge_index[1].astype(jnp.int32)
    N, D = x.shape
    E = src.shape[0]
    H = D
    NP = -(-N // 2560) * 2560
    EPAD = -(-E // (NW * CH_A * 2)) * (NW * CH_A * 2)
    EP = EPAD
    assert EP % BE == 0

    starts_all = jnp.searchsorted(
        dst, jnp.arange(N, dtype=jnp.int32)).astype(jnp.int32)
    counts = jnp.diff(jnp.concatenate(
        [starts_all, jnp.array([E], jnp.int32)]))
    T = counts.max().astype(jnp.int32)

    order = jnp.argsort(-counts).astype(jnp.int32)
    counts_s = jnp.concatenate(
        [counts[order], jnp.zeros((NP - N,), jnp.int32)])
    starts_s = jnp.concatenate(
        [starts_all[order], jnp.full((NP - N,), E - 1, jnp.int32)])
    counts_asc = counts_s[::-1]
    pos = jnp.argsort(order).astype(jnp.int32)
    pos_src = pos[src]
    pos_dst = pos[dst]
    x_s = jnp.concatenate([x[order], jnp.zeros((NP - N, D), jnp.float32)])

    def pad_idx(ix):
        return jnp.concatenate(
            [ix, jnp.full((EPAD - E,), N, jnp.int32)]
        ).reshape(NW, EPAD // NW // CH_A, CH_A)

    idx_src = pad_idx(pos_src)
    idx_dst = pad_idx(pos_dst)

    gather_rows = _make_gather_rows(NP, H, EPAD)
    step_gather = _make_step_gather(EPAD, H, NP, E)
    lstm_step = _make_lstm_step(NP, H)
    tail = _make_tail(NP, H, N)

    def layer(h_in, p):
        wihT = p['W_ih'].T
        whhT = p['W_hh'].T
        b = (p['b_ih'] + p['b_hh']).reshape(1, 4 * H)

        feats = gather_rows(h_in, idx_src)

        def cond(carry):
            t, _, _ = carry
            return t < T

        def body(carry):
            t, h, c = carry
            kt = (NP - jnp.searchsorted(counts_asc, t, side='right')
                  ).astype(jnp.int32)
            tk = jnp.full((16,), t, jnp.int32)
            xt = step_gather(feats, starts_s, counts_s, tk)
            h, c = lstm_step(kt.reshape(1), xt, h, c, wihT, whhT, b)
            return t + 1, h, c

        z = jnp.zeros((NP, H), jnp.float32)
        _, hl, _ = lax.while_loop(cond, body, (jnp.int32(0), z, z))
        return tail(hl, h_in, p['W_l'].T, p['W_r'].T, p['b_l'].reshape(1, H))

    h = layer(x_s, params['conv1'])
    h = layer(h, params['conv2'])
    h = layer(h, params['conv3'])

    hs = gather_rows(h, idx_src)
    hd = gather_rows(h, idx_dst)
    m = params['edge_mlp']
    H2 = m['W2'].shape[0]
    OUT = m['W3'].shape[0]
    ED = edge_attr.shape[1]
    w1 = m['W1'].T

    ea = jnp.concatenate(
        [edge_attr.astype(jnp.float32),
         jnp.zeros((EP - E, ED), jnp.float32)])

    mlp = _make_mlp(EP, H, ED, H2, OUT)
    out = mlp(hs, hd, ea, w1[:H], w1[H:2 * H], w1[2 * H:],
              m['b1'].reshape(1, H),
              m['W2'].T, m['b2'].reshape(1, H2),
              m['W3'].T, m['b3'].reshape(1, OUT))
    return out[:E]
